# Initial kernel scaffold; baseline (speedup 1.0000x reference)
import jax
import jax.numpy as jnp
from jax.experimental import pallas as pl


def kernel(x, wg, w1, b1, w2, b2):
    raise NotImplementedError("write your pallas kernel here")


# speedup vs baseline: 1.3657x; 1.3657x over previous
import dataclasses
import functools

import jax
import jax.numpy as jnp
from jax.experimental import pallas as pl
from jax.experimental.pallas import tpu as pltpu
from jax.experimental.pallas import tpu_sc as plsc

_T = 2048
_D = 1024
_E = 8
_F = 2048
_K = 2
_C = (_T * _K) // _E
_EC = _E * _C

_NUM_WORKERS = 32
_ROWS_PER_CHUNK = 32


def _cumsum_lanes(a):
    n = a.shape[1]
    k = 1
    while k < n:
        shifted = jnp.concatenate(
            [jnp.zeros((a.shape[0], k), a.dtype), a[:, :-k]], axis=1)
        a = a + shifted
        k *= 2
    return a


def _gate_body(x_ref, wg_ref, fi_ref, kf_ref, wc_ref):
    x = x_ref[...]
    wg = wg_ref[...]
    logits = jnp.dot(x, wg, preferred_element_type=jnp.float32)
    lt = logits.T
    mx = jnp.max(lt, axis=0, keepdims=True)
    eg = jnp.exp(lt - mx)
    gates = eg / jnp.sum(eg, axis=0, keepdims=True)

    iota = jax.lax.broadcasted_iota(jnp.int32, (_E, _T), 0)
    m1 = jnp.max(gates, axis=0, keepdims=True)
    idx1 = jnp.min(jnp.where(gates == m1, iota, _E), axis=0, keepdims=True)
    g2 = jnp.where(iota == idx1, -jnp.inf, gates)
    m2 = jnp.max(g2, axis=0, keepdims=True)
    idx2 = jnp.min(jnp.where(g2 == m2, iota, _E), axis=0, keepdims=True)

    s = m1 + m2 + 1e-9
    w1v = m1 / s
    w2v = m2 / s

    oh1 = (iota == idx1).astype(jnp.int32)
    oh2 = (iota == idx2).astype(jnp.int32)
    inc1 = _cumsum_lanes(oh1)
    inc2 = _cumsum_lanes(oh2)
    cnt1 = inc1[:, _T - 1:_T]
    pos1 = jnp.sum(jnp.where(oh1 == 1, inc1 - 1, 0), axis=0, keepdims=True)
    pos2 = jnp.sum(jnp.where(oh2 == 1, inc2 - 1 + cnt1, 0), axis=0,
                   keepdims=True)

    keep1 = pos1 < _C
    keep2 = pos2 < _C
    fi1 = jnp.where(keep1, idx1 * _C + pos1, _EC - 1)
    fi2 = jnp.where(keep2, idx2 * _C + pos2, _EC - 1)

    fi_ref[...] = jnp.concatenate([fi1, fi2], axis=0)
    kf_ref[...] = jnp.concatenate(
        [keep1.astype(jnp.int32), keep2.astype(jnp.int32)], axis=0)
    wcT = jnp.concatenate([jnp.where(keep1, w1v, 0.0),
                           jnp.where(keep2, w2v, 0.0)], axis=0)
    wc_ref[...] = wcT.T


def _gate(x, wg, interpret=False):
    return pl.pallas_call(
        _gate_body,
        out_shape=(jax.ShapeDtypeStruct((_K, _T), jnp.int32),
                   jax.ShapeDtypeStruct((_K, _T), jnp.int32),
                   jax.ShapeDtypeStruct((_T, _K), jnp.float32)),
        interpret=interpret,
    )(x, wg)


def _sc_compiler_params():
    cp = pltpu.CompilerParams()
    if "needs_layout_passes" in pltpu.CompilerParams.__dataclass_fields__:
        cp = dataclasses.replace(cp, needs_layout_passes=False)
    return cp


def _make_inv_kernel():
    mesh = plsc.VectorSubcoreMesh(core_axis_name="c", subcore_axis_name="s")

    @functools.partial(
        pl.kernel,
        out_type=jax.ShapeDtypeStruct((_EC,), jnp.int32),
        mesh=mesh,
        compiler_params=_sc_compiler_params(),
        scratch_types=[pltpu.VMEM((_EC,), jnp.int32),
                       pltpu.VMEM((_EC,), jnp.int32),
                       pltpu.VMEM((_EC,), jnp.int32)],
    )
    def inv_kernel(fi_hbm, kf_hbm, inv_hbm, fi_v, kf_v, inv_v):
        cid = jax.lax.axis_index("c")
        sid = jax.lax.axis_index("s")

        @pl.when(jnp.logical_and(cid == 0, sid == 0))
        def _():
            pltpu.sync_copy(fi_hbm, fi_v)
            pltpu.sync_copy(kf_hbm, kf_v)

            @pl.loop(0, _EC // 16)
            def _(i):
                inv_v[pl.ds(i * 16, 16)] = jnp.zeros((16,), jnp.int32)

            @pl.loop(0, _EC // 16)
            def _(i):
                base = i * 16
                f = fi_v[pl.ds(base, 16)]
                kf = kf_v[pl.ds(base, 16)]
                tb = jnp.where(base >= _T, base - _T, base)
                tok = jax.lax.iota(jnp.int32, 16) + tb
                plsc.store_scatter(inv_v, [f], tok, mask=kf == 1)

            pltpu.sync_copy(inv_v, inv_hbm)

    return inv_kernel


def _build_inv(fi_flat, kf_flat):
    return _make_inv_kernel()(fi_flat, kf_flat)


def _make_gather_kernel(table_rows):
    mesh = plsc.VectorSubcoreMesh(core_axis_name="c", subcore_axis_name="s")
    per_worker = _EC // _NUM_WORKERS
    n_chunks = per_worker // _ROWS_PER_CHUNK

    @functools.partial(
        pl.kernel,
        out_type=jax.ShapeDtypeStruct((_EC, _D), jnp.float32),
        mesh=mesh,
        scratch_types=[pltpu.VMEM((_ROWS_PER_CHUNK,), jnp.int32),
                       pltpu.VMEM((_ROWS_PER_CHUNK, _D), jnp.float32),
                       pltpu.SemaphoreType.DMA],
    )
    def gather_kernel(table_hbm, idx_hbm, out_hbm, idx_v, rows_v, sem):
        wid = jax.lax.axis_index("s") * 2 + jax.lax.axis_index("c")
        base = wid * per_worker

        @pl.loop(0, n_chunks)
        def _(ci):
            off = base + ci * _ROWS_PER_CHUNK
            pltpu.sync_copy(idx_hbm.at[pl.ds(off, _ROWS_PER_CHUNK)], idx_v)
            pltpu.async_copy(table_hbm.at[idx_v], rows_v, sem).wait()
            pltpu.sync_copy(rows_v, out_hbm.at[pl.ds(off, _ROWS_PER_CHUNK)])

    return gather_kernel


def _gather_rows(table, idx):
    return _make_gather_kernel(table.shape[0])(table, idx)


def _ffn_body(xin_ref, w1_ref, b1_ref, w2_ref, b2_ref, out_ref):
    xin = xin_ref[0]
    h = jnp.maximum(
        jnp.dot(xin, w1_ref[0], preferred_element_type=jnp.float32)
        + b1_ref[0], 0.0)
    out_ref[0] = (jnp.dot(h, w2_ref[0], preferred_element_type=jnp.float32)
                  + b2_ref[0])


def _ffn(xin3, w1, b1r, w2, b2r, interpret=False):
    return pl.pallas_call(
        _ffn_body,
        grid=(_E,),
        in_specs=[
            pl.BlockSpec((1, _C, _D), lambda e: (e, 0, 0)),
            pl.BlockSpec((1, _D, _F), lambda e: (e, 0, 0)),
            pl.BlockSpec((1, 1, _F), lambda e: (e, 0, 0)),
            pl.BlockSpec((1, _F, _D), lambda e: (e, 0, 0)),
            pl.BlockSpec((1, 1, _D), lambda e: (e, 0, 0)),
        ],
        out_specs=pl.BlockSpec((1, _C, _D), lambda e: (e, 0, 0)),
        out_shape=jax.ShapeDtypeStruct((_E, _C, _D), jnp.float32),
        interpret=interpret,
    )(xin3, w1, b1r, w2, b2r)


def _combine_body(g_ref, wc_ref, y_ref):
    g0 = g_ref[0]
    g1 = g_ref[1]
    w0 = wc_ref[:, 0:1]
    w1 = wc_ref[:, 1:2]
    y_ref[...] = (jnp.where(w0 > 0, w0 * g0, 0.0)
                  + jnp.where(w1 > 0, w1 * g1, 0.0))


def _combine(g3, wc, interpret=False):
    return pl.pallas_call(
        _combine_body,
        out_shape=jax.ShapeDtypeStruct((_T, _D), jnp.float32),
        interpret=interpret,
    )(g3, wc)


def kernel(x, wg, w1, b1, w2, b2):
    fi, kf, wc = _gate(x, wg)
    fi_flat = fi.reshape(_K * _T)
    kf_flat = kf.reshape(_K * _T)
    inv = _build_inv(fi_flat, kf_flat)
    buf = _gather_rows(x, inv)
    eo = _ffn(buf.reshape(_E, _C, _D), w1, b1.reshape(_E, 1, _F),
              w2, b2.reshape(_E, 1, _D)).reshape(_EC, _D)
    g = _gather_rows(eo, fi_flat)
    return _combine(g.reshape(_K, _T, _D), wc)


# speedup vs baseline: 1.4086x; 1.0315x over previous
import dataclasses
import functools

import jax
import jax.numpy as jnp
from jax.experimental import pallas as pl
from jax.experimental.pallas import tpu as pltpu
from jax.experimental.pallas import tpu_sc as plsc

_T = 2048
_D = 1024
_E = 8
_F = 2048
_K = 2
_C = (_T * _K) // _E
_EC = _E * _C

_NUM_WORKERS = 32
_ROWS_PER_CHUNK = 32


def _cumsum_lanes(a):
    n = a.shape[1]
    k = 1
    while k < n:
        shifted = jnp.concatenate(
            [jnp.zeros((a.shape[0], k), a.dtype), a[:, :-k]], axis=1)
        a = a + shifted
        k *= 2
    return a


def _gate_body(x_ref, wg_ref, fi_ref, kf_ref, wc_ref):
    x = x_ref[...]
    wg = wg_ref[...]
    logits = jnp.dot(x, wg, preferred_element_type=jnp.float32)
    lt = logits.T
    mx = jnp.max(lt, axis=0, keepdims=True)
    eg = jnp.exp(lt - mx)
    gates = eg / jnp.sum(eg, axis=0, keepdims=True)

    iota = jax.lax.broadcasted_iota(jnp.int32, (_E, _T), 0)
    m1 = jnp.max(gates, axis=0, keepdims=True)
    idx1 = jnp.min(jnp.where(gates == m1, iota, _E), axis=0, keepdims=True)
    g2 = jnp.where(iota == idx1, -jnp.inf, gates)
    m2 = jnp.max(g2, axis=0, keepdims=True)
    idx2 = jnp.min(jnp.where(g2 == m2, iota, _E), axis=0, keepdims=True)

    s = m1 + m2 + 1e-9
    w1v = m1 / s
    w2v = m2 / s

    oh1 = (iota == idx1).astype(jnp.int32)
    oh2 = (iota == idx2).astype(jnp.int32)
    inc1 = _cumsum_lanes(oh1)
    inc2 = _cumsum_lanes(oh2)
    cnt1 = inc1[:, _T - 1:_T]
    pos1 = jnp.sum(jnp.where(oh1 == 1, inc1 - 1, 0), axis=0, keepdims=True)
    pos2 = jnp.sum(jnp.where(oh2 == 1, inc2 - 1 + cnt1, 0), axis=0,
                   keepdims=True)

    keep1 = pos1 < _C
    keep2 = pos2 < _C
    fi1 = jnp.where(keep1, idx1 * _C + pos1, _EC - 1)
    fi2 = jnp.where(keep2, idx2 * _C + pos2, _EC - 1)

    fi_ref[...] = jnp.concatenate([fi1, fi2], axis=0)
    kf_ref[...] = jnp.concatenate(
        [keep1.astype(jnp.int32), keep2.astype(jnp.int32)], axis=0)
    wcT = jnp.concatenate([jnp.where(keep1, w1v, 0.0),
                           jnp.where(keep2, w2v, 0.0)], axis=0)
    wc_ref[...] = wcT.T


def _gate(x, wg, interpret=False):
    return pl.pallas_call(
        _gate_body,
        out_shape=(jax.ShapeDtypeStruct((_K, _T), jnp.int32),
                   jax.ShapeDtypeStruct((_K, _T), jnp.int32),
                   jax.ShapeDtypeStruct((_T, _K), jnp.float32)),
        interpret=interpret,
    )(x, wg)


def _sc_compiler_params():
    cp = pltpu.CompilerParams()
    if "needs_layout_passes" in pltpu.CompilerParams.__dataclass_fields__:
        cp = dataclasses.replace(cp, needs_layout_passes=False)
    return cp


def _mesh():
    return plsc.VectorSubcoreMesh(core_axis_name="c", subcore_axis_name="s")


def _make_dispatch_kernel():
    per_worker = _EC // _NUM_WORKERS
    ch = _ROWS_PER_CHUNK
    n_chunks = per_worker // ch

    @functools.partial(
        pl.kernel,
        out_type=jax.ShapeDtypeStruct((_EC, _D), jnp.float32),
        mesh=_mesh(),
        compiler_params=_sc_compiler_params(),
        scratch_types=[pltpu.VMEM((_EC,), jnp.int32),
                       pltpu.VMEM((_EC,), jnp.int32),
                       pltpu.VMEM((_EC,), jnp.int32),
                       pltpu.VMEM((ch, _D), jnp.float32),
                       pltpu.VMEM((ch, _D), jnp.float32),
                       pltpu.SemaphoreType.DMA,
                       pltpu.SemaphoreType.DMA,
                       pltpu.SemaphoreType.DMA,
                       pltpu.SemaphoreType.DMA],
    )
    def dispatch_kernel(x_hbm, fi_hbm, kf_hbm, buf_hbm,
                        fi_v, kf_v, inv_v, b0, b1, gs0, gs1, ws0, ws1):
        wid = jax.lax.axis_index("s") * 2 + jax.lax.axis_index("c")
        base = wid * per_worker
        pltpu.sync_copy(fi_hbm, fi_v)
        pltpu.sync_copy(kf_hbm, kf_v)

        @pl.loop(0, per_worker // 16)
        def _(i):
            inv_v[pl.ds(base + i * 16, 16)] = jnp.zeros((16,), jnp.int32)

        @pl.loop(0, _EC // 16)
        def _(i):
            eb = i * 16
            f = fi_v[pl.ds(eb, 16)]
            kf = kf_v[pl.ds(eb, 16)]
            tb = jnp.where(eb >= _T, eb - _T, eb)
            tok = jax.lax.iota(jnp.int32, 16) + tb
            plsc.store_scatter(inv_v, [f], tok, mask=kf == 1)

        my_idx = inv_v.at[pl.ds(base, per_worker)]
        bufs = (b0, b1)
        gsems = (gs0, gs1)
        wsems = (ws0, ws1)
        writes = [None, None]
        gathers = [None, None]
        for c in range(n_chunks):
            s = c % 2
            if writes[s] is not None:
                writes[s].wait()
            gathers[s] = pltpu.async_copy(
                x_hbm.at[my_idx.at[pl.ds(c * ch, ch)]], bufs[s], gsems[s])
            if c % 2 == 1:
                for s2 in (0, 1):
                    gathers[s2].wait()
                    writes[s2] = pltpu.async_copy(
                        bufs[s2],
                        buf_hbm.at[pl.ds(base + (c - 1 + s2) * ch, ch)],
                        wsems[s2])
        for s2 in (0, 1):
            if writes[s2] is not None:
                writes[s2].wait()

    return dispatch_kernel


def _dispatch(x, fi_flat, kf_flat):
    return _make_dispatch_kernel()(x, fi_flat, kf_flat)


def _make_combine_gather_kernel():
    per_worker = _EC // _NUM_WORKERS
    ch = _ROWS_PER_CHUNK
    n_chunks = per_worker // ch

    @functools.partial(
        pl.kernel,
        out_type=jax.ShapeDtypeStruct((_EC, _D), jnp.float32),
        mesh=_mesh(),
        compiler_params=_sc_compiler_params(),
        scratch_types=[pltpu.VMEM((per_worker,), jnp.int32),
                       pltpu.VMEM((ch, _D), jnp.float32),
                       pltpu.VMEM((ch, _D), jnp.float32),
                       pltpu.SemaphoreType.DMA,
                       pltpu.SemaphoreType.DMA,
                       pltpu.SemaphoreType.DMA,
                       pltpu.SemaphoreType.DMA],
    )
    def combine_kernel(eo_hbm, fi_hbm, out_hbm,
                       idx_v, b0, b1, gs0, gs1, ws0, ws1):
        wid = jax.lax.axis_index("s") * 2 + jax.lax.axis_index("c")
        base = wid * per_worker
        pltpu.sync_copy(fi_hbm.at[pl.ds(base, per_worker)], idx_v)
        bufs = (b0, b1)
        gsems = (gs0, gs1)
        wsems = (ws0, ws1)
        writes = [None, None]
        gathers = [None, None]
        for c in range(n_chunks):
            s = c % 2
            if writes[s] is not None:
                writes[s].wait()
            gathers[s] = pltpu.async_copy(
                eo_hbm.at[idx_v.at[pl.ds(c * ch, ch)]], bufs[s], gsems[s])
            if c % 2 == 1:
                for s2 in (0, 1):
                    gathers[s2].wait()
                    writes[s2] = pltpu.async_copy(
                        bufs[s2],
                        out_hbm.at[pl.ds(base + (c - 1 + s2) * ch, ch)],
                        wsems[s2])
        for s2 in (0, 1):
            if writes[s2] is not None:
                writes[s2].wait()

    return combine_kernel


def _gather_rows(table, idx):
    return _make_combine_gather_kernel()(table, idx)


def _ffn_body(xin_ref, w1_ref, b1_ref, w2_ref, b2_ref, out_ref):
    xin = xin_ref[0]
    h = jnp.maximum(
        jnp.dot(xin, w1_ref[0], preferred_element_type=jnp.float32)
        + b1_ref[0], 0.0)
    out_ref[0] = (jnp.dot(h, w2_ref[0], preferred_element_type=jnp.float32)
                  + b2_ref[0])


def _ffn(xin3, w1, b1r, w2, b2r, interpret=False):
    return pl.pallas_call(
        _ffn_body,
        grid=(_E,),
        in_specs=[
            pl.BlockSpec((1, _C, _D), lambda e: (e, 0, 0)),
            pl.BlockSpec((1, _D, _F), lambda e: (e, 0, 0)),
            pl.BlockSpec((1, 1, _F), lambda e: (e, 0, 0)),
            pl.BlockSpec((1, _F, _D), lambda e: (e, 0, 0)),
            pl.BlockSpec((1, 1, _D), lambda e: (e, 0, 0)),
        ],
        out_specs=pl.BlockSpec((1, _C, _D), lambda e: (e, 0, 0)),
        out_shape=jax.ShapeDtypeStruct((_E, _C, _D), jnp.float32),
        interpret=interpret,
    )(xin3, w1, b1r, w2, b2r)


def _combine_body(g_ref, wc_ref, y_ref):
    g0 = g_ref[0]
    g1 = g_ref[1]
    w0 = wc_ref[:, 0:1]
    w1 = wc_ref[:, 1:2]
    y_ref[...] = (jnp.where(w0 > 0, w0 * g0, 0.0)
                  + jnp.where(w1 > 0, w1 * g1, 0.0))


def _combine(g3, wc, interpret=False):
    return pl.pallas_call(
        _combine_body,
        out_shape=jax.ShapeDtypeStruct((_T, _D), jnp.float32),
        interpret=interpret,
    )(g3, wc)


def kernel(x, wg, w1, b1, w2, b2):
    fi, kf, wc = _gate(x, wg)
    fi_flat = fi.reshape(_K * _T)
    kf_flat = kf.reshape(_K * _T)
    buf = _dispatch(x, fi_flat, kf_flat)
    eo = _ffn(buf.reshape(_E, _C, _D), w1, b1.reshape(_E, 1, _F),
              w2, b2.reshape(_E, 1, _D)).reshape(_EC, _D)
    g = _gather_rows(eo, fi_flat)
    return _combine(g.reshape(_K, _T, _D), wc)


# speedup vs baseline: 1.4097x; 1.0008x over previous
import dataclasses
import functools

import jax
import jax.numpy as jnp
from jax.experimental import pallas as pl
from jax.experimental.pallas import tpu as pltpu
from jax.experimental.pallas import tpu_sc as plsc

_T = 2048
_D = 1024
_E = 8
_F = 2048
_K = 2
_C = (_T * _K) // _E
_EC = _E * _C

_NUM_WORKERS = 32
_ROWS_PER_CHUNK = 32


def _cumsum_lanes(a):
    n = a.shape[1]
    k = 1
    while k < n:
        shifted = jnp.concatenate(
            [jnp.zeros((a.shape[0], k), a.dtype), a[:, :-k]], axis=1)
        a = a + shifted
        k *= 2
    return a


def _gate_body(x_ref, wg_ref, fi_ref, kf_ref, wc_ref):
    x = x_ref[...]
    wg = wg_ref[...]
    logits = jnp.dot(x, wg, preferred_element_type=jnp.float32)
    lt = logits.T
    mx = jnp.max(lt, axis=0, keepdims=True)
    eg = jnp.exp(lt - mx)
    gates = eg / jnp.sum(eg, axis=0, keepdims=True)

    iota = jax.lax.broadcasted_iota(jnp.int32, (_E, _T), 0)
    m1 = jnp.max(gates, axis=0, keepdims=True)
    idx1 = jnp.min(jnp.where(gates == m1, iota, _E), axis=0, keepdims=True)
    g2 = jnp.where(iota == idx1, -jnp.inf, gates)
    m2 = jnp.max(g2, axis=0, keepdims=True)
    idx2 = jnp.min(jnp.where(g2 == m2, iota, _E), axis=0, keepdims=True)

    s = m1 + m2 + 1e-9
    w1v = m1 / s
    w2v = m2 / s

    oh1 = (iota == idx1).astype(jnp.int32)
    oh2 = (iota == idx2).astype(jnp.int32)
    inc1 = _cumsum_lanes(oh1)
    inc2 = _cumsum_lanes(oh2)
    cnt1 = inc1[:, _T - 1:_T]
    pos1 = jnp.sum(jnp.where(oh1 == 1, inc1 - 1, 0), axis=0, keepdims=True)
    pos2 = jnp.sum(jnp.where(oh2 == 1, inc2 - 1 + cnt1, 0), axis=0,
                   keepdims=True)

    keep1 = pos1 < _C
    keep2 = pos2 < _C
    fi1 = jnp.where(keep1, idx1 * _C + pos1, _EC - 1)
    fi2 = jnp.where(keep2, idx2 * _C + pos2, _EC - 1)

    fi_ref[...] = jnp.concatenate([fi1, fi2], axis=0)
    kf_ref[...] = jnp.concatenate(
        [keep1.astype(jnp.int32), keep2.astype(jnp.int32)], axis=0)
    wcT = jnp.concatenate([jnp.where(keep1, w1v, 0.0),
                           jnp.where(keep2, w2v, 0.0)], axis=0)
    wc_ref[...] = wcT.T


def _gate(x, wg, interpret=False):
    return pl.pallas_call(
        _gate_body,
        out_shape=(jax.ShapeDtypeStruct((_K, _T), jnp.int32),
                   jax.ShapeDtypeStruct((_K, _T), jnp.int32),
                   jax.ShapeDtypeStruct((_T, _K), jnp.float32)),
        interpret=interpret,
    )(x, wg)


def _sc_compiler_params():
    cp = pltpu.CompilerParams()
    if "needs_layout_passes" in pltpu.CompilerParams.__dataclass_fields__:
        cp = dataclasses.replace(cp, needs_layout_passes=False)
    return cp


def _mesh():
    return plsc.VectorSubcoreMesh(core_axis_name="c", subcore_axis_name="s")


def _make_dispatch_kernel():
    per_worker = _EC // _NUM_WORKERS
    ch = _ROWS_PER_CHUNK
    n_chunks = per_worker // ch

    @functools.partial(
        pl.kernel,
        out_type=jax.ShapeDtypeStruct((_EC, _D), jnp.float32),
        mesh=_mesh(),
        compiler_params=_sc_compiler_params(),
        scratch_types=[pltpu.VMEM((_EC,), jnp.int32),
                       pltpu.VMEM((_EC,), jnp.int32),
                       pltpu.VMEM((_EC,), jnp.int32),
                       pltpu.VMEM((ch, _D), jnp.float32),
                       pltpu.VMEM((ch, _D), jnp.float32),
                       pltpu.SemaphoreType.DMA,
                       pltpu.SemaphoreType.DMA,
                       pltpu.SemaphoreType.DMA,
                       pltpu.SemaphoreType.DMA],
    )
    def dispatch_kernel(x_hbm, fi_hbm, kf_hbm, buf_hbm,
                        fi_v, kf_v, inv_v, b0, b1, gs0, gs1, ws0, ws1):
        wid = jax.lax.axis_index("s") * 2 + jax.lax.axis_index("c")
        base = wid * per_worker
        pltpu.sync_copy(fi_hbm, fi_v)
        pltpu.sync_copy(kf_hbm, kf_v)

        @pl.loop(0, per_worker // 16)
        def _(i):
            inv_v[pl.ds(base + i * 16, 16)] = jnp.zeros((16,), jnp.int32)

        @pl.loop(0, _EC // 16)
        def _(i):
            eb = i * 16
            f = fi_v[pl.ds(eb, 16)]
            kf = kf_v[pl.ds(eb, 16)]
            tb = jnp.where(eb >= _T, eb - _T, eb)
            tok = jax.lax.iota(jnp.int32, 16) + tb
            plsc.store_scatter(inv_v, [f], tok, mask=kf == 1)

        my_idx = inv_v.at[pl.ds(base, per_worker)]
        bufs = (b0, b1)
        gsems = (gs0, gs1)
        wsems = (ws0, ws1)
        writes = [None, None]
        gathers = [None, None]
        for c in range(n_chunks):
            s = c % 2
            if writes[s] is not None:
                writes[s].wait()
            gathers[s] = pltpu.async_copy(
                x_hbm.at[my_idx.at[pl.ds(c * ch, ch)]], bufs[s], gsems[s])
            if c % 2 == 1:
                for s2 in (0, 1):
                    gathers[s2].wait()
                    writes[s2] = pltpu.async_copy(
                        bufs[s2],
                        buf_hbm.at[pl.ds(base + (c - 1 + s2) * ch, ch)],
                        wsems[s2])
        for s2 in (0, 1):
            if writes[s2] is not None:
                writes[s2].wait()

    return dispatch_kernel


def _dispatch(x, fi_flat, kf_flat):
    return _make_dispatch_kernel()(x, fi_flat, kf_flat)


def _make_combine_gather_kernel():
    per_worker = _EC // _NUM_WORKERS
    ch = _ROWS_PER_CHUNK
    n_chunks = per_worker // ch

    @functools.partial(
        pl.kernel,
        out_type=jax.ShapeDtypeStruct((_EC, _D), jnp.float32),
        mesh=_mesh(),
        compiler_params=_sc_compiler_params(),
        scratch_types=[pltpu.VMEM((per_worker,), jnp.int32),
                       pltpu.VMEM((ch, _D), jnp.float32),
                       pltpu.VMEM((ch, _D), jnp.float32),
                       pltpu.SemaphoreType.DMA,
                       pltpu.SemaphoreType.DMA,
                       pltpu.SemaphoreType.DMA,
                       pltpu.SemaphoreType.DMA],
    )
    def combine_kernel(eo_hbm, fi_hbm, out_hbm,
                       idx_v, b0, b1, gs0, gs1, ws0, ws1):
        wid = jax.lax.axis_index("s") * 2 + jax.lax.axis_index("c")
        base = wid * per_worker
        pltpu.sync_copy(fi_hbm.at[pl.ds(base, per_worker)], idx_v)
        bufs = (b0, b1)
        gsems = (gs0, gs1)
        wsems = (ws0, ws1)
        writes = [None, None]
        gathers = [None, None]
        for c in range(n_chunks):
            s = c % 2
            if writes[s] is not None:
                writes[s].wait()
            gathers[s] = pltpu.async_copy(
                eo_hbm.at[idx_v.at[pl.ds(c * ch, ch)]], bufs[s], gsems[s])
            if c % 2 == 1:
                for s2 in (0, 1):
                    gathers[s2].wait()
                    writes[s2] = pltpu.async_copy(
                        bufs[s2],
                        out_hbm.at[pl.ds(base + (c - 1 + s2) * ch, ch)],
                        wsems[s2])
        for s2 in (0, 1):
            if writes[s2] is not None:
                writes[s2].wait()

    return combine_kernel


def _gather_rows(table, idx):
    return _make_combine_gather_kernel()(table, idx)


def _ffn_body(xin_ref, w1_ref, b1_ref, w2_ref, b2_ref, out_ref):
    xin = xin_ref[0].astype(jnp.bfloat16)
    h = jnp.maximum(
        jnp.dot(xin, w1_ref[0].astype(jnp.bfloat16),
                preferred_element_type=jnp.float32)
        + b1_ref[0], 0.0).astype(jnp.bfloat16)
    out_ref[0] = (jnp.dot(h, w2_ref[0].astype(jnp.bfloat16),
                          preferred_element_type=jnp.float32)
                  + b2_ref[0])


def _ffn(xin3, w1, b1r, w2, b2r, interpret=False):
    return pl.pallas_call(
        _ffn_body,
        grid=(_E,),
        in_specs=[
            pl.BlockSpec((1, _C, _D), lambda e: (e, 0, 0)),
            pl.BlockSpec((1, _D, _F), lambda e: (e, 0, 0)),
            pl.BlockSpec((1, 1, _F), lambda e: (e, 0, 0)),
            pl.BlockSpec((1, _F, _D), lambda e: (e, 0, 0)),
            pl.BlockSpec((1, 1, _D), lambda e: (e, 0, 0)),
        ],
        out_specs=pl.BlockSpec((1, _C, _D), lambda e: (e, 0, 0)),
        out_shape=jax.ShapeDtypeStruct((_E, _C, _D), jnp.float32),
        interpret=interpret,
    )(xin3, w1, b1r, w2, b2r)


def _combine_body(g_ref, wc_ref, y_ref):
    g0 = g_ref[0]
    g1 = g_ref[1]
    w0 = wc_ref[:, 0:1]
    w1 = wc_ref[:, 1:2]
    y_ref[...] = (jnp.where(w0 > 0, w0 * g0, 0.0)
                  + jnp.where(w1 > 0, w1 * g1, 0.0))


def _combine(g3, wc, interpret=False):
    return pl.pallas_call(
        _combine_body,
        out_shape=jax.ShapeDtypeStruct((_T, _D), jnp.float32),
        interpret=interpret,
    )(g3, wc)


def kernel(x, wg, w1, b1, w2, b2):
    fi, kf, wc = _gate(x, wg)
    fi_flat = fi.reshape(_K * _T)
    kf_flat = kf.reshape(_K * _T)
    buf = _dispatch(x, fi_flat, kf_flat)
    eo = _ffn(buf.reshape(_E, _C, _D), w1, b1.reshape(_E, 1, _F),
              w2, b2.reshape(_E, 1, _D)).reshape(_EC, _D)
    g = _gather_rows(eo, fi_flat)
    return _combine(g.reshape(_K, _T, _D), wc)
